# Initial kernel scaffold; baseline (speedup 1.0000x reference)
import jax
import jax.numpy as jnp
from jax.experimental import pallas as pl


def kernel(x, V):
    raise NotImplementedError("write your pallas kernel here")


# speedup vs baseline: 32.3517x; 32.3517x over previous
import functools

import numpy as np
import jax
import jax.numpy as jnp
from jax import lax
from jax.experimental import pallas as pl
from jax.experimental.pallas import tpu as pltpu
from jax.experimental.pallas import tpu_sc as plsc

OUT_DIM = 5
BASIS = OUT_DIM * 16

_rs = np.random.RandomState(0)
HASH_A = [int(v) for v in _rs.randint(1, 2**31 - 1, size=(OUT_DIM,)).astype(np.int32)]
HASH_B = [int(v) for v in _rs.randint(0, 2**31 - 1, size=(OUT_DIM,)).astype(np.int32)]

BATCH = 16384
NUM_CORES = 2
NUM_SUBCORES = 16
NW = NUM_CORES * NUM_SUBCORES
BPW = BATCH // NW
LANES = 16
CHUNKS = BPW // LANES


def _body(x_hbm, v_hbm, out_hbm, x_v, v_v, out_v, sem):
    wid = lax.axis_index("s") * NUM_CORES + lax.axis_index("c")
    base = wid * BPW
    cx = pltpu.async_copy(x_hbm.at[pl.ds(base, BPW)], x_v, sem)
    cv = pltpu.async_copy(v_hbm, v_v.at[pl.ds(0, BASIS)], sem)
    cx.wait()
    cv.wait()

    def chunk(i):
        xv = x_v[pl.ds(i * LANES, LANES)]
        embs = []
        s = None
        for d in range(OUT_DIM):
            h = xv * jnp.int32(HASH_A[d]) + jnp.int32(HASH_B[d])
            a = h & jnp.int32(15)
            g = lax.shift_right_arithmetic(h, jnp.int32(4))
            sign = lax.shift_right_logical(g, jnp.int32(31))
            t = (
                (g & jnp.int32(255))
                + (lax.shift_right_logical(g, jnp.int32(8)) & jnp.int32(255))
                + (lax.shift_right_logical(g, jnp.int32(16)) & jnp.int32(255))
                + lax.shift_right_logical(g, jnp.int32(24))
            )
            t = t - sign
            t2 = (t & jnp.int32(255)) + lax.shift_right_logical(t, jnp.int32(8))
            q = lax.shift_right_logical(t2 * jnp.int32(205), jnp.int32(10))
            c = t2 - jnp.int32(5) * q
            r = a + lax.shift_left(c, jnp.int32(4))
            e = plsc.load_gather(v_v, [r])
            embs.append(e)
            s = e if s is None else s + e
        scale = jnp.float32(OUT_DIM) / s
        for d in range(OUT_DIM):
            out_v[d, pl.ds(i * LANES, LANES)] = embs[d] * scale

    @plsc.parallel_loop(0, CHUNKS, unroll=1)
    def _loop(i):
        chunk(i)

    pltpu.async_copy(out_v, out_hbm.at[:, pl.ds(base, BPW)], sem).wait()


@functools.cache
def _sc_call():
    return pl.kernel(
        _body,
        mesh=plsc.VectorSubcoreMesh(
            core_axis_name="c", subcore_axis_name="s",
            num_cores=NUM_CORES, num_subcores=NUM_SUBCORES),
        compiler_params=pltpu.CompilerParams(needs_layout_passes=False),
        out_type=jax.ShapeDtypeStruct((OUT_DIM, BATCH), jnp.float32),
        name="hash_embed_sc",
        scratch_types=[
            pltpu.VMEM((BPW,), jnp.int32),
            pltpu.VMEM((128,), jnp.float32),
            pltpu.VMEM((OUT_DIM, BPW), jnp.float32),
            pltpu.SemaphoreType.DMA,
        ],
    )


def kernel(x, V):
    return _sc_call()(x, V).T
